# baseline (speedup 1.0000x reference)
import functools

import jax
import jax.numpy as jnp
from jax import lax
from jax.experimental import pallas as pl
from jax.experimental.pallas import tpu as pltpu
from jax.experimental.pallas import tpu_sc as plsc

N = 10000
NE = 320000
D_IN = 128
D_HID = 32

NW = 32
C = 128
KC0 = 88
KC1 = 72
KC = KC0 + KC1
EP = 16 * KC * C
NP = 10112
RPT = NP // 16
NBUF = 8

_mesh = plsc.VectorSubcoreMesh(core_axis_name="c", subcore_axis_name="s")
_params = pltpu.CompilerParams(use_tc_tiling_on_sc=False)


def _deg_body(dst_hbm, out_hbm, dst_v, ones_v, z_v, acc_sh, sem):
    cid = lax.axis_index("c")
    sid = lax.axis_index("s")
    for i in range(C // 16):
        ones_v[pl.ds(i * 16, 16)] = jnp.ones((16,), jnp.float32)

    def zfill(i, c):
        z_v[pl.ds(i * 16, 16)] = jnp.zeros((16,), jnp.float32)
        return c

    lax.fori_loop(0, RPT // 16 + 1, zfill, 0)
    pltpu.sync_copy(z_v.at[pl.ds(0, RPT)], acc_sh.at[pl.ds(sid * RPT, RPT)])

    @pl.when(cid == 0)
    def _():
        pltpu.sync_copy(dst_hbm.at[pl.ds(sid * KC0, KC0)],
                        dst_v.at[pl.ds(0, KC0)])

    @pl.when(cid == 1)
    def _():
        pltpu.sync_copy(dst_hbm.at[pl.ds(16 * KC0 + sid * KC1, KC1)],
                        dst_v.at[pl.ds(0, KC1)])

    plsc.subcore_barrier()

    def scatter_all(kc):
        def fire(j, c):
            pltpu.async_copy(ones_v, acc_sh.at[dst_v.at[j]], sem, add=True)
            return c

        lax.fori_loop(0, kc, fire, 0)

        def drain(j, c):
            pltpu.make_async_copy(ones_v, acc_sh.at[dst_v.at[0]], sem).wait()
            return c

        lax.fori_loop(0, kc, drain, 0)

    @pl.when(cid == 0)
    def _():
        scatter_all(KC0)

    @pl.when(cid == 1)
    def _():
        scatter_all(KC1)

    plsc.subcore_barrier()
    pltpu.sync_copy(acc_sh.at[pl.ds(sid * RPT, RPT)], z_v.at[pl.ds(0, RPT)])
    pltpu.sync_copy(z_v.at[pl.ds(0, RPT)],
                    out_hbm.at[pl.ds(cid * NP + sid * RPT, RPT)])


_deg_call = functools.partial(
    pl.kernel, _deg_body,
    mesh=_mesh,
    compiler_params=_params,
    out_type=jax.ShapeDtypeStruct((2 * NP,), jnp.float32),
    scratch_types=[
        pltpu.VMEM((max(KC0, KC1), C), jnp.int32),
        pltpu.VMEM((C,), jnp.float32),
        pltpu.VMEM((RPT + 16,), jnp.float32),
        pltpu.VMEM_SHARED((NP,), jnp.float32),
        pltpu.SemaphoreType.DMA,
    ],
)()


def _seg_body(src_hbm, dst_hbm, t_hbm, zeros_hbm, out_hbm,
              src_v, dst_v, rows_v, acc_sh, t_sh, sems):
    cid = lax.axis_index("c")
    sid = lax.axis_index("s")
    pltpu.sync_copy(zeros_hbm.at[pl.ds(sid * RPT, RPT)],
                    acc_sh.at[pl.ds(sid * RPT, RPT)])
    pltpu.sync_copy(t_hbm.at[pl.ds(sid * RPT, RPT)],
                    t_sh.at[pl.ds(sid * RPT, RPT)])

    @pl.when(cid == 0)
    def _():
        pltpu.sync_copy(src_hbm.at[pl.ds(sid * KC0, KC0)],
                        src_v.at[pl.ds(0, KC0)])
        pltpu.sync_copy(dst_hbm.at[pl.ds(sid * KC0, KC0)],
                        dst_v.at[pl.ds(0, KC0)])

    @pl.when(cid == 1)
    def _():
        pltpu.sync_copy(src_hbm.at[pl.ds(16 * KC0 + sid * KC1, KC1)],
                        src_v.at[pl.ds(0, KC1)])
        pltpu.sync_copy(dst_hbm.at[pl.ds(16 * KC0 + sid * KC1, KC1)],
                        dst_v.at[pl.ds(0, KC1)])

    plsc.subcore_barrier()

    def gather(j, b):
        return pltpu.make_async_copy(t_sh.at[src_v.at[j]], rows_v.at[b],
                                     sems.at[b])

    def ring(kc):
        for b in range(NBUF):
            gather(b, b).start()

        def group(g, c):
            j0 = g * NBUF
            for b in range(NBUF):
                j = j0 + b
                gather(j, b).wait()
                pltpu.sync_copy(rows_v.at[b], acc_sh.at[dst_v.at[j]], add=True)

                @pl.when(j + NBUF < kc)
                def _():
                    gather(j + NBUF, b).start()
            return c

        lax.fori_loop(0, kc // NBUF, group, 0)

    @pl.when(cid == 0)
    def _():
        ring(KC0)

    @pl.when(cid == 1)
    def _():
        ring(KC1)

    plsc.subcore_barrier()
    pltpu.sync_copy(acc_sh.at[pl.ds(sid * RPT, RPT)],
                    out_hbm.at[pl.ds(cid * NP + sid * RPT, RPT)])


_seg_call = functools.partial(
    pl.kernel, _seg_body,
    mesh=_mesh,
    compiler_params=_params,
    out_type=jax.ShapeDtypeStruct((2 * NP, D_HID), jnp.bfloat16),
    scratch_types=[
        pltpu.VMEM((max(KC0, KC1), C), jnp.int32),
        pltpu.VMEM((max(KC0, KC1), C), jnp.int32),
        pltpu.VMEM((NBUF, C, D_HID), jnp.bfloat16),
        pltpu.VMEM_SHARED((NP, D_HID), jnp.bfloat16),
        pltpu.VMEM_SHARED((NP, D_HID), jnp.bfloat16),
        pltpu.SemaphoreType.DMA((NBUF,)),
    ],
)()


def _mm_body(x_ref, w_ref, o_ref):
    o_ref[:N] = jnp.dot(x_ref[...], w_ref[...],
                        preferred_element_type=jnp.float32)
    o_ref[N:] = jnp.zeros((NP - N, D_HID), jnp.float32)


def _scale_body(xw_ref, d0_ref, d1_ref, t_ref):
    dinv = lax.rsqrt(d0_ref[...] + d1_ref[...] + 1.0)
    t_ref[...] = (xw_ref[...] * dinv).astype(jnp.bfloat16)


def _finish_body(s_ref, t_ref, d0_ref, d1_ref, b_ref,
                 wmu_ref, bmu_ref, wlv_ref, blv_ref, mu_ref, lv_ref):
    agg = (s_ref[:NP].astype(jnp.float32) + s_ref[NP:].astype(jnp.float32)
           + t_ref[...].astype(jnp.float32))
    dinv = lax.rsqrt(d0_ref[...] + d1_ref[...] + 1.0)
    h = jnp.maximum(agg * dinv + b_ref[...], 0.0)
    rows = lax.broadcasted_iota(jnp.int32, (NP, 1), 0)
    h = jnp.where(rows < N, h, 0.0)
    hm = jnp.sum(h, axis=0, keepdims=True) * (1.0 / N)
    mu_ref[...] = (jnp.dot(hm, wmu_ref[...], preferred_element_type=jnp.float32)
                   + bmu_ref[...])
    lv_ref[...] = (jnp.dot(hm, wlv_ref[...], preferred_element_type=jnp.float32)
                   + blv_ref[...])


def kernel(x, edge_index, W, b, Wmu, bmu, Wlv, blv):
    ei = edge_index.astype(jnp.int32)
    pad_e = EP - NE
    src = jnp.concatenate([ei[0], jnp.full((pad_e,), N, jnp.int32)])
    dst = jnp.concatenate([ei[1], jnp.full((pad_e,), N, jnp.int32)])
    src2 = src.reshape(16 * KC, C)
    dst2 = dst.reshape(16 * KC, C)
    zeros2 = jnp.zeros((NP, D_HID), jnp.bfloat16)

    deg_p = _deg_call(dst2)
    d0 = deg_p[:NP].reshape(NP, 1)
    d1 = deg_p[NP:].reshape(NP, 1)

    xw = pl.pallas_call(
        _mm_body,
        out_shape=jax.ShapeDtypeStruct((NP, D_HID), jnp.float32),
    )(x, W)

    t = pl.pallas_call(
        _scale_body,
        out_shape=jax.ShapeDtypeStruct((NP, D_HID), jnp.bfloat16),
    )(xw, d0, d1)

    s_p = _seg_call(src2, dst2, t, zeros2)

    mu, lv = pl.pallas_call(
        _finish_body,
        out_shape=(jax.ShapeDtypeStruct((1, D_HID), jnp.float32),
                   jax.ShapeDtypeStruct((1, D_HID), jnp.float32)),
    )(s_p, t, d0, d1, b.reshape(1, D_HID),
      Wmu, bmu.reshape(1, D_HID), Wlv, blv.reshape(1, D_HID))
    return (mu.reshape(D_HID), lv.reshape(D_HID))


# speedup vs baseline: 1.0069x; 1.0069x over previous
import functools

import jax
import jax.numpy as jnp
from jax import lax
from jax.experimental import pallas as pl
from jax.experimental.pallas import tpu as pltpu
from jax.experimental.pallas import tpu_sc as plsc

N = 10000
NE = 320000
D_IN = 128
D_HID = 32

NW = 32
C = 128
KC0 = 84
KC1 = 76
KC = KC0 + KC1
EP = 16 * KC * C
NP = 10112
RPT = NP // 16
NBUF = 4

_mesh = plsc.VectorSubcoreMesh(core_axis_name="c", subcore_axis_name="s")
_params = pltpu.CompilerParams(use_tc_tiling_on_sc=False)


def _deg_body(dst_hbm, out_hbm, dst_v, ones_v, z_v, acc_sh, sem):
    cid = lax.axis_index("c")
    sid = lax.axis_index("s")
    for i in range(C // 16):
        ones_v[pl.ds(i * 16, 16)] = jnp.ones((16,), jnp.float32)

    def zfill(i, c):
        z_v[pl.ds(i * 16, 16)] = jnp.zeros((16,), jnp.float32)
        return c

    lax.fori_loop(0, RPT // 16 + 1, zfill, 0)
    pltpu.sync_copy(z_v.at[pl.ds(0, RPT)], acc_sh.at[pl.ds(sid * RPT, RPT)])

    @pl.when(cid == 0)
    def _():
        pltpu.sync_copy(dst_hbm.at[pl.ds(sid * KC0, KC0)],
                        dst_v.at[pl.ds(0, KC0)])

    @pl.when(cid == 1)
    def _():
        pltpu.sync_copy(dst_hbm.at[pl.ds(16 * KC0 + sid * KC1, KC1)],
                        dst_v.at[pl.ds(0, KC1)])

    plsc.subcore_barrier()

    def scatter_all(kc):
        def fire(j, c):
            pltpu.async_copy(ones_v, acc_sh.at[dst_v.at[j]], sem, add=True)
            return c

        lax.fori_loop(0, kc, fire, 0)

        def drain(j, c):
            pltpu.make_async_copy(ones_v, acc_sh.at[dst_v.at[0]], sem).wait()
            return c

        lax.fori_loop(0, kc, drain, 0)

    @pl.when(cid == 0)
    def _():
        scatter_all(KC0)

    @pl.when(cid == 1)
    def _():
        scatter_all(KC1)

    plsc.subcore_barrier()
    pltpu.sync_copy(acc_sh.at[pl.ds(sid * RPT, RPT)], z_v.at[pl.ds(0, RPT)])
    pltpu.sync_copy(z_v.at[pl.ds(0, RPT)],
                    out_hbm.at[pl.ds(cid * NP + sid * RPT, RPT)])


_deg_call = functools.partial(
    pl.kernel, _deg_body,
    mesh=_mesh,
    compiler_params=_params,
    out_type=jax.ShapeDtypeStruct((2 * NP,), jnp.float32),
    scratch_types=[
        pltpu.VMEM((max(KC0, KC1), C), jnp.int32),
        pltpu.VMEM((C,), jnp.float32),
        pltpu.VMEM((RPT + 16,), jnp.float32),
        pltpu.VMEM_SHARED((NP,), jnp.float32),
        pltpu.SemaphoreType.DMA,
    ],
)()


def _seg_body(src_hbm, dst_hbm, t_hbm, zeros_hbm, out_hbm,
              src_v, dst_v, rows_v, acc_sh, t_sh, sems):
    cid = lax.axis_index("c")
    sid = lax.axis_index("s")
    pltpu.sync_copy(zeros_hbm.at[pl.ds(sid * RPT, RPT)],
                    acc_sh.at[pl.ds(sid * RPT, RPT)])
    pltpu.sync_copy(t_hbm.at[pl.ds(sid * RPT, RPT)],
                    t_sh.at[pl.ds(sid * RPT, RPT)])

    @pl.when(cid == 0)
    def _():
        pltpu.sync_copy(src_hbm.at[pl.ds(sid * KC0, KC0)],
                        src_v.at[pl.ds(0, KC0)])
        pltpu.sync_copy(dst_hbm.at[pl.ds(sid * KC0, KC0)],
                        dst_v.at[pl.ds(0, KC0)])

    @pl.when(cid == 1)
    def _():
        pltpu.sync_copy(src_hbm.at[pl.ds(16 * KC0 + sid * KC1, KC1)],
                        src_v.at[pl.ds(0, KC1)])
        pltpu.sync_copy(dst_hbm.at[pl.ds(16 * KC0 + sid * KC1, KC1)],
                        dst_v.at[pl.ds(0, KC1)])

    plsc.subcore_barrier()

    def gather(j, b):
        return pltpu.make_async_copy(t_sh.at[src_v.at[j]], rows_v.at[b],
                                     sems.at[b])

    def ring(kc):
        for b in range(NBUF):
            gather(b, b).start()

        def group(g, c):
            j0 = g * NBUF
            for b in range(NBUF):
                j = j0 + b
                gather(j, b).wait()
                pltpu.sync_copy(rows_v.at[b], acc_sh.at[dst_v.at[j]], add=True)

                @pl.when(j + NBUF < kc)
                def _():
                    gather(j + NBUF, b).start()
            return c

        lax.fori_loop(0, kc // NBUF, group, 0)

    @pl.when(cid == 0)
    def _():
        ring(KC0)

    @pl.when(cid == 1)
    def _():
        ring(KC1)

    plsc.subcore_barrier()
    pltpu.sync_copy(acc_sh.at[pl.ds(sid * RPT, RPT)],
                    out_hbm.at[pl.ds(cid * NP + sid * RPT, RPT)])


_seg_call = functools.partial(
    pl.kernel, _seg_body,
    mesh=_mesh,
    compiler_params=_params,
    out_type=jax.ShapeDtypeStruct((2 * NP, D_HID), jnp.bfloat16),
    scratch_types=[
        pltpu.VMEM((max(KC0, KC1), C), jnp.int32),
        pltpu.VMEM((max(KC0, KC1), C), jnp.int32),
        pltpu.VMEM((NBUF, C, D_HID), jnp.bfloat16),
        pltpu.VMEM_SHARED((NP, D_HID), jnp.bfloat16),
        pltpu.VMEM_SHARED((NP, D_HID), jnp.bfloat16),
        pltpu.SemaphoreType.DMA((NBUF,)),
    ],
)()


def _mm_body(x_ref, w_ref, o_ref):
    o_ref[:N] = jnp.dot(x_ref[...], w_ref[...],
                        preferred_element_type=jnp.float32)
    o_ref[N:] = jnp.zeros((NP - N, D_HID), jnp.float32)


def _scale_body(xw_ref, d0_ref, d1_ref, t_ref):
    dinv = lax.rsqrt(d0_ref[...] + d1_ref[...] + 1.0)
    t_ref[...] = (xw_ref[...] * dinv).astype(jnp.bfloat16)


def _finish_body(s_ref, t_ref, d0_ref, d1_ref, b_ref,
                 wmu_ref, bmu_ref, wlv_ref, blv_ref, mu_ref, lv_ref):
    agg = (s_ref[:NP].astype(jnp.float32) + s_ref[NP:].astype(jnp.float32)
           + t_ref[...].astype(jnp.float32))
    dinv = lax.rsqrt(d0_ref[...] + d1_ref[...] + 1.0)
    h = jnp.maximum(agg * dinv + b_ref[...], 0.0)
    rows = lax.broadcasted_iota(jnp.int32, (NP, 1), 0)
    h = jnp.where(rows < N, h, 0.0)
    hm = jnp.sum(h, axis=0, keepdims=True) * (1.0 / N)
    mu_ref[...] = (jnp.dot(hm, wmu_ref[...], preferred_element_type=jnp.float32)
                   + bmu_ref[...])
    lv_ref[...] = (jnp.dot(hm, wlv_ref[...], preferred_element_type=jnp.float32)
                   + blv_ref[...])


def kernel(x, edge_index, W, b, Wmu, bmu, Wlv, blv):
    ei = edge_index.astype(jnp.int32)
    pad_e = EP - NE
    src = jnp.concatenate([ei[0], jnp.full((pad_e,), N, jnp.int32)])
    dst = jnp.concatenate([ei[1], jnp.full((pad_e,), N, jnp.int32)])
    src2 = src.reshape(16 * KC, C)
    dst2 = dst.reshape(16 * KC, C)
    zeros2 = jnp.zeros((NP, D_HID), jnp.bfloat16)

    deg_p = _deg_call(dst2)
    d0 = deg_p[:NP].reshape(NP, 1)
    d1 = deg_p[NP:].reshape(NP, 1)

    xw = pl.pallas_call(
        _mm_body,
        out_shape=jax.ShapeDtypeStruct((NP, D_HID), jnp.float32),
    )(x, W)

    t = pl.pallas_call(
        _scale_body,
        out_shape=jax.ShapeDtypeStruct((NP, D_HID), jnp.bfloat16),
    )(xw, d0, d1)

    s_p = _seg_call(src2, dst2, t, zeros2)

    mu, lv = pl.pallas_call(
        _finish_body,
        out_shape=(jax.ShapeDtypeStruct((1, D_HID), jnp.float32),
                   jax.ShapeDtypeStruct((1, D_HID), jnp.float32)),
    )(s_p, t, d0, d1, b.reshape(1, D_HID),
      Wmu, bmu.reshape(1, D_HID), Wlv, blv.reshape(1, D_HID))
    return (mu.reshape(D_HID), lv.reshape(D_HID))


# speedup vs baseline: 1.0081x; 1.0012x over previous
import functools

import jax
import jax.numpy as jnp
from jax import lax
from jax.experimental import pallas as pl
from jax.experimental.pallas import tpu as pltpu
from jax.experimental.pallas import tpu_sc as plsc

N = 10000
NE = 320000
D_IN = 128
D_HID = 32

NW = 32
C = 128
KC0 = 84
KC1 = 76
KC = KC0 + KC1
EP = 16 * KC * C
NP = 10112
RPT = NP // 16
NBUF = 4

_mesh = plsc.VectorSubcoreMesh(core_axis_name="c", subcore_axis_name="s")
_params = pltpu.CompilerParams(use_tc_tiling_on_sc=False)


def _deg_body(dst_hbm, out_hbm, dst_v, ones_v, z_v, acc_sh, sem):
    cid = lax.axis_index("c")
    sid = lax.axis_index("s")
    for i in range(C // 16):
        ones_v[pl.ds(i * 16, 16)] = jnp.ones((16,), jnp.float32)

    def zfill(i, c):
        z_v[pl.ds(i * 16, 16)] = jnp.zeros((16,), jnp.float32)
        return c

    lax.fori_loop(0, RPT // 16 + 1, zfill, 0)
    pltpu.sync_copy(z_v.at[pl.ds(0, RPT)], acc_sh.at[pl.ds(sid * RPT, RPT)])

    @pl.when(cid == 0)
    def _():
        pltpu.sync_copy(dst_hbm.at[pl.ds(sid * KC0, KC0)],
                        dst_v.at[pl.ds(0, KC0)])

    @pl.when(cid == 1)
    def _():
        pltpu.sync_copy(dst_hbm.at[pl.ds(16 * KC0 + sid * KC1, KC1)],
                        dst_v.at[pl.ds(0, KC1)])

    plsc.subcore_barrier()

    def scatter_all(kc):
        def fire(j, c):
            pltpu.async_copy(ones_v, acc_sh.at[dst_v.at[j]], sem, add=True)
            return c

        lax.fori_loop(0, kc, fire, 0)

        def drain(j, c):
            pltpu.make_async_copy(ones_v, acc_sh.at[dst_v.at[0]], sem).wait()
            return c

        lax.fori_loop(0, kc, drain, 0)

    @pl.when(cid == 0)
    def _():
        scatter_all(KC0)

    @pl.when(cid == 1)
    def _():
        scatter_all(KC1)

    plsc.subcore_barrier()
    pltpu.sync_copy(acc_sh.at[pl.ds(sid * RPT, RPT)], z_v.at[pl.ds(0, RPT)])
    pltpu.sync_copy(z_v.at[pl.ds(0, RPT)],
                    out_hbm.at[pl.ds(cid * NP + sid * RPT, RPT)])


_deg_call = functools.partial(
    pl.kernel, _deg_body,
    mesh=_mesh,
    compiler_params=_params,
    out_type=jax.ShapeDtypeStruct((2 * NP,), jnp.float32),
    scratch_types=[
        pltpu.VMEM((max(KC0, KC1), C), jnp.int32),
        pltpu.VMEM((C,), jnp.float32),
        pltpu.VMEM((RPT + 16,), jnp.float32),
        pltpu.VMEM_SHARED((NP,), jnp.float32),
        pltpu.SemaphoreType.DMA,
    ],
)()


def _seg_body(src_hbm, dst_hbm, t_hbm, zeros_hbm, out_hbm,
              src_v, dst_v, rows_v, acc_sh, t_sh, sems):
    cid = lax.axis_index("c")
    sid = lax.axis_index("s")
    pltpu.sync_copy(zeros_hbm.at[pl.ds(sid * RPT, RPT)],
                    acc_sh.at[pl.ds(sid * RPT, RPT)])
    pltpu.sync_copy(t_hbm.at[pl.ds(sid * RPT, RPT)],
                    t_sh.at[pl.ds(sid * RPT, RPT)])

    @pl.when(cid == 0)
    def _():
        pltpu.sync_copy(src_hbm.at[pl.ds(sid * KC0, KC0)],
                        src_v.at[pl.ds(0, KC0)])
        pltpu.sync_copy(dst_hbm.at[pl.ds(sid * KC0, KC0)],
                        dst_v.at[pl.ds(0, KC0)])

    @pl.when(cid == 1)
    def _():
        pltpu.sync_copy(src_hbm.at[pl.ds(16 * KC0 + sid * KC1, KC1)],
                        src_v.at[pl.ds(0, KC1)])
        pltpu.sync_copy(dst_hbm.at[pl.ds(16 * KC0 + sid * KC1, KC1)],
                        dst_v.at[pl.ds(0, KC1)])

    plsc.subcore_barrier()

    def gather(j, b):
        return pltpu.make_async_copy(t_sh.at[src_v.at[j]], rows_v.at[b],
                                     sems.at[b])

    def ring(kc):
        for b in range(NBUF):
            gather(b, b).start()

        def group(g, c):
            j0 = g * NBUF
            for b in range(NBUF):
                j = j0 + b
                gather(j, b).wait()
                pltpu.sync_copy(rows_v.at[b], acc_sh.at[dst_v.at[j]], add=True)

                @pl.when(j + NBUF < kc)
                def _():
                    gather(j + NBUF, b).start()
            return c

        lax.fori_loop(0, kc // NBUF, group, 0)

    @pl.when(cid == 0)
    def _():
        ring(KC0)

    @pl.when(cid == 1)
    def _():
        ring(KC1)

    plsc.subcore_barrier()
    pltpu.sync_copy(acc_sh.at[pl.ds(sid * RPT, RPT)],
                    out_hbm.at[pl.ds(cid * NP + sid * RPT, RPT)])


_seg_call = functools.partial(
    pl.kernel, _seg_body,
    mesh=_mesh,
    compiler_params=_params,
    out_type=jax.ShapeDtypeStruct((2 * NP, D_HID), jnp.bfloat16),
    scratch_types=[
        pltpu.VMEM((max(KC0, KC1), C), jnp.int32),
        pltpu.VMEM((max(KC0, KC1), C), jnp.int32),
        pltpu.VMEM((NBUF, C, D_HID), jnp.bfloat16),
        pltpu.VMEM_SHARED((NP, D_HID), jnp.bfloat16),
        pltpu.VMEM_SHARED((NP, D_HID), jnp.bfloat16),
        pltpu.SemaphoreType.DMA((NBUF,)),
    ],
)()


def _mmscale_body(x_ref, w_ref, d0_ref, d1_ref, t_ref):
    xw = jnp.dot(x_ref[...], w_ref[...], preferred_element_type=jnp.float32)
    dinv = lax.rsqrt(d0_ref[:N] + d1_ref[:N] + 1.0)
    t_ref[:N] = (xw * dinv).astype(jnp.bfloat16)
    t_ref[N:] = jnp.zeros((NP - N, D_HID), jnp.bfloat16)


def _finish_body(s_ref, t_ref, d0_ref, d1_ref, b_ref,
                 wmu_ref, bmu_ref, wlv_ref, blv_ref, mu_ref, lv_ref):
    agg = (s_ref[:NP].astype(jnp.float32) + s_ref[NP:].astype(jnp.float32)
           + t_ref[...].astype(jnp.float32))
    dinv = lax.rsqrt(d0_ref[...] + d1_ref[...] + 1.0)
    h = jnp.maximum(agg * dinv + b_ref[...], 0.0)
    rows = lax.broadcasted_iota(jnp.int32, (NP, 1), 0)
    h = jnp.where(rows < N, h, 0.0)
    hm = jnp.sum(h, axis=0, keepdims=True) * (1.0 / N)
    mu_ref[...] = (jnp.dot(hm, wmu_ref[...], preferred_element_type=jnp.float32)
                   + bmu_ref[...])
    lv_ref[...] = (jnp.dot(hm, wlv_ref[...], preferred_element_type=jnp.float32)
                   + blv_ref[...])


def kernel(x, edge_index, W, b, Wmu, bmu, Wlv, blv):
    ei = edge_index.astype(jnp.int32)
    pad_e = EP - NE
    src = jnp.concatenate([ei[0], jnp.full((pad_e,), N, jnp.int32)])
    dst = jnp.concatenate([ei[1], jnp.full((pad_e,), N, jnp.int32)])
    src2 = src.reshape(16 * KC, C)
    dst2 = dst.reshape(16 * KC, C)
    zeros2 = jnp.zeros((NP, D_HID), jnp.bfloat16)

    deg_p = _deg_call(dst2)
    d0 = deg_p[:NP].reshape(NP, 1)
    d1 = deg_p[NP:].reshape(NP, 1)

    t = pl.pallas_call(
        _mmscale_body,
        out_shape=jax.ShapeDtypeStruct((NP, D_HID), jnp.bfloat16),
    )(x, W, d0, d1)

    s_p = _seg_call(src2, dst2, t, zeros2)

    mu, lv = pl.pallas_call(
        _finish_body,
        out_shape=(jax.ShapeDtypeStruct((1, D_HID), jnp.float32),
                   jax.ShapeDtypeStruct((1, D_HID), jnp.float32)),
    )(s_p, t, d0, d1, b.reshape(1, D_HID),
      Wmu, bmu.reshape(1, D_HID), Wlv, blv.reshape(1, D_HID))
    return (mu.reshape(D_HID), lv.reshape(D_HID))
